# baseline (speedup 1.0000x reference)
import functools

import jax
import jax.numpy as jnp
from jax import lax
from jax.experimental import pallas as pl
from jax.experimental.pallas import tpu as pltpu
from jax.experimental.pallas import tpu_sc as plsc

N = 10000
E = 320000
D = 128
L = 5
G = 64
BN_EPS = 1e-5

NW = 32
EW = E // NW
CH = 40
NCH = 256
WIN = 8
NWIN = NCH // WIN
NBUF = 8
LA = NBUF - 1
NPAD = 10112
RPT = NPAD // 16
DUMMY = N

BLK = 1000
NB = N // BLK

def _sc_body(h_hbm, src_hbm, dst_hbm, zt_hbm, out_hbm, *refs):
    sv = refs[0:2]
    dv = refs[2:4]
    bufs = refs[4:4 + NBUF]
    agg = refs[4 + NBUF]
    gsem = refs[5 + NBUF:5 + 2 * NBUF]
    asem = refs[5 + 2 * NBUF:5 + 3 * NBUF]
    isem = refs[5 + 3 * NBUF:7 + 3 * NBUF]
    jsem = refs[7 + 3 * NBUF:9 + 3 * NBUF]
    sv0, sv1 = sv
    dv0, dv1 = dv
    c = lax.axis_index("c")
    s = lax.axis_index("s")
    wk = c * 16 + s
    base = s * RPT
    pltpu.sync_copy(zt_hbm, agg.at[pl.ds(base, RPT)])
    pltpu.sync_copy(src_hbm.at[wk, pl.ds(0, WIN)], sv0)
    pltpu.sync_copy(src_hbm.at[wk, pl.ds(WIN, WIN)], sv1)
    pltpu.sync_copy(dst_hbm.at[wk, pl.ds(0, WIN)], dv0)
    pltpu.sync_copy(dst_hbm.at[wk, pl.ds(WIN, WIN)], dv1)
    plsc.subcore_barrier()

    for b in range(LA):
        pltpu.async_copy(h_hbm.at[sv0.at[b]], bufs[b], gsem[b])

    def window(w, p):
        for k in range(WIN):
            ch = w * WIN + k
            b = k % NBUF
            b3 = (k + LA) % NBUF
            pltpu.make_async_copy(
                h_hbm.at[sv[p].at[k]], bufs[b], gsem[b]).wait()
            if k == 0:
                @pl.when(w > 0)
                def _():
                    pltpu.make_async_copy(
                        bufs[b3], agg.at[dv[p].at[0]], asem[b3]).wait()

                @pl.when(jnp.logical_and(w > 0, w + 1 < NWIN))
                def _():
                    pltpu.async_copy(
                        src_hbm.at[wk, pl.ds((w + 1) * WIN, WIN)],
                        sv[1 - p], jsem[1 - p])
                    pltpu.async_copy(
                        dst_hbm.at[wk, pl.ds((w + 1) * WIN, WIN)],
                        dv[1 - p], isem[1 - p])

                @pl.when(w > 1)
                def _():
                    pltpu.make_async_copy(
                        dst_hbm.at[wk, pl.ds(w * WIN, WIN)], dv[p],
                        isem[p]).wait()
            else:
                pltpu.make_async_copy(
                    bufs[b3], agg.at[dv[p].at[k - 1]], asem[b3]).wait()

            if k == WIN - LA or (LA > WIN and k == 1):
                @pl.when(jnp.logical_and(w > 0, w + 1 < NWIN))
                def _():
                    pltpu.make_async_copy(
                        src_hbm.at[wk, pl.ds((w + 1) * WIN, WIN)],
                        sv[1 - p], jsem[1 - p]).wait()

            if k + LA < WIN:
                pltpu.async_copy(h_hbm.at[sv[p].at[k + LA]], bufs[b3],
                                 gsem[b3])
            else:
                @pl.when(w + 1 < NWIN)
                def _():
                    pltpu.async_copy(
                        h_hbm.at[sv[1 - p].at[k + LA - WIN]], bufs[b3],
                        gsem[b3])

            pltpu.async_copy(bufs[b], agg.at[dv[p].at[k]], asem[b], add=True)

    def wpair(wp, carry):
        window(wp * 2, 0)
        window(wp * 2 + 1, 1)
        return carry

    lax.fori_loop(0, NWIN // 2, wpair, 0)
    pltpu.make_async_copy(
        bufs[(NCH - 1) % NBUF], agg.at[dv1.at[WIN - 1]],
        asem[(NCH - 1) % NBUF]).wait()
    plsc.subcore_barrier()
    pltpu.sync_copy(agg.at[pl.ds(base, RPT)], out_hbm.at[c, pl.ds(base, RPT)])


@functools.cache
def _sc_seg_kernel():
    mesh = plsc.VectorSubcoreMesh(core_axis_name="c", subcore_axis_name="s")
    return pl.kernel(
        _sc_body,
        out_type=jax.ShapeDtypeStruct((2, NPAD, D), jnp.float32),
        mesh=mesh,
        scratch_types=(
            [pltpu.VMEM((WIN, CH), jnp.int32) for _ in range(4)]
            + [pltpu.VMEM((CH, D), jnp.float32) for _ in range(NBUF)]
            + [pltpu.VMEM_SHARED((NPAD, D), jnp.float32)]
            + [pltpu.SemaphoreType.DMA] * (2 * NBUF + 4)
        ),
    )


def _sc_seg(h, src3, dst3, zt):
    return _sc_seg_kernel()(h, src3, dst3, zt)


def _k1(h_ref, p0_ref, p1_ref, w_ref, b_ref, z_ref, st_ref):
    i = pl.program_id(0)
    sv = h_ref[...] + p0_ref[...] + p1_ref[...]
    z = jnp.dot(sv, w_ref[...], preferred_element_type=jnp.float32) + b_ref[...]
    z_ref[...] = z
    cs = jnp.sum(z, axis=0, keepdims=True)
    cq = jnp.sum(z * z, axis=0, keepdims=True)
    contrib = jnp.concatenate([cs, cq, jnp.zeros((6, D), jnp.float32)], axis=0)

    @pl.when(i == 0)
    def _():
        st_ref[...] = contrib

    @pl.when(i != 0)
    def _():
        st_ref[...] += contrib


def _stage1(h, p0, p1, W1, b1):
    return pl.pallas_call(
        _k1,
        grid=(NB,),
        in_specs=[
            pl.BlockSpec((BLK, D), lambda i: (i, 0)),
            pl.BlockSpec((BLK, D), lambda i: (i, 0)),
            pl.BlockSpec((BLK, D), lambda i: (i, 0)),
            pl.BlockSpec((D, D), lambda i: (0, 0)),
            pl.BlockSpec((1, D), lambda i: (0, 0)),
        ],
        out_specs=[
            pl.BlockSpec((BLK, D), lambda i: (i, 0)),
            pl.BlockSpec((8, D), lambda i: (0, 0)),
        ],
        out_shape=[
            jax.ShapeDtypeStruct((N, D), jnp.float32),
            jax.ShapeDtypeStruct((8, D), jnp.float32),
        ],
    )(h, p0, p1, W1, b1)


def _kpool(h_ref, bt_ref, pool_ref):
    i = pl.program_id(0)
    b = bt_ref[0, 0, :]
    oh = (lax.broadcasted_iota(jnp.int32, (G, BLK), 0) == b[None, :]).astype(
        jnp.float32)
    pc = jnp.dot(oh, h_ref[...], preferred_element_type=jnp.float32)

    @pl.when(i == 0)
    def _():
        pool_ref[...] = pc

    @pl.when(i != 0)
    def _():
        pool_ref[...] += pc


def _pool(h, bt3):
    return pl.pallas_call(
        _kpool,
        grid=(NB,),
        in_specs=[
            pl.BlockSpec((BLK, D), lambda i: (i, 0)),
            pl.BlockSpec((1, 1, BLK), lambda i: (i, 0, 0)),
        ],
        out_specs=pl.BlockSpec((G, D), lambda i: (0, 0)),
        out_shape=jax.ShapeDtypeStruct((G, D), jnp.float32),
    )(h, bt3)


def _k2(z_ref, st_ref, w_ref, b_ref, g_ref, bb_ref, o_ref, st2_ref):
    i = pl.program_id(0)
    st = st_ref[...]
    mu = st[0:1, :] * (1.0 / N)
    var = st[1:2, :] * (1.0 / N) - mu * mu
    scale = g_ref[...] * lax.rsqrt(var + BN_EPS)
    r = jnp.maximum((z_ref[...] - mu) * scale + bb_ref[...], 0.0)
    z2 = jnp.dot(r, w_ref[...], preferred_element_type=jnp.float32) + b_ref[...]
    o_ref[...] = z2
    cs = jnp.sum(z2, axis=0, keepdims=True)
    cq = jnp.sum(z2 * z2, axis=0, keepdims=True)
    contrib = jnp.concatenate([cs, cq, jnp.zeros((6, D), jnp.float32)], axis=0)

    @pl.when(i == 0)
    def _():
        st2_ref[...] = contrib

    @pl.when(i != 0)
    def _():
        st2_ref[...] += contrib


def _stage2(z1, st1, W2, b2, g1, bb1):
    return pl.pallas_call(
        _k2,
        grid=(NB,),
        in_specs=[
            pl.BlockSpec((BLK, D), lambda i: (i, 0)),
            pl.BlockSpec((8, D), lambda i: (0, 0)),
            pl.BlockSpec((D, D), lambda i: (0, 0)),
            pl.BlockSpec((1, D), lambda i: (0, 0)),
            pl.BlockSpec((1, D), lambda i: (0, 0)),
            pl.BlockSpec((1, D), lambda i: (0, 0)),
        ],
        out_specs=[
            pl.BlockSpec((BLK, D), lambda i: (i, 0)),
            pl.BlockSpec((8, D), lambda i: (0, 0)),
        ],
        out_shape=[
            jax.ShapeDtypeStruct((N, D), jnp.float32),
            jax.ShapeDtypeStruct((8, D), jnp.float32),
        ],
    )(z1, st1, W2, b2, g1, bb1)


def _k3(z_ref, st_ref, g_ref, bb_ref, h_ref):
    st = st_ref[...]
    mu = st[0:1, :] * (1.0 / N)
    var = st[1:2, :] * (1.0 / N) - mu * mu
    scale = g_ref[...] * lax.rsqrt(var + BN_EPS)
    h_ref[...] = jnp.maximum((z_ref[...] - mu) * scale + bb_ref[...], 0.0)


def _stage3(z2, st2, g, bb):
    return pl.pallas_call(
        _k3,
        grid=(NB,),
        in_specs=[
            pl.BlockSpec((BLK, D), lambda i: (i, 0)),
            pl.BlockSpec((8, D), lambda i: (0, 0)),
            pl.BlockSpec((1, D), lambda i: (0, 0)),
            pl.BlockSpec((1, D), lambda i: (0, 0)),
        ],
        out_specs=pl.BlockSpec((BLK, D), lambda i: (i, 0)),
        out_shape=jax.ShapeDtypeStruct((N, D), jnp.float32),
    )(z2, st2, g, bb)


def _k4(h_ref, bt_ref, ps_ref, fw_ref, fb_ref, o_ref, acc_ref, cnt_ref):
    i = pl.program_id(0)

    @pl.when(i == 0)
    def _():
        acc_ref[...] = jnp.zeros_like(acc_ref)
        cnt_ref[...] = jnp.zeros_like(cnt_ref)

    b = bt_ref[0, 0, :]
    oh = (lax.broadcasted_iota(jnp.int32, (G, BLK), 0) == b[None, :]).astype(
        jnp.float32)
    acc_ref[...] += jnp.dot(oh, h_ref[...], preferred_element_type=jnp.float32)
    cnt_ref[...] += jnp.dot(oh, jnp.ones((BLK, D), jnp.float32),
                            preferred_element_type=jnp.float32)

    @pl.when(i == NB - 1)
    def _():
        invc = 1.0 / jnp.maximum(cnt_ref[...], 1.0)
        out = jnp.dot(acc_ref[...] * invc, fw_ref[L],
                      preferred_element_type=jnp.float32)
        for k in range(L):
            out += jnp.dot(ps_ref[k * G:(k + 1) * G, :] * invc, fw_ref[k],
                           preferred_element_type=jnp.float32)
        out += jnp.sum(fb_ref[...], axis=0, keepdims=True)
        o_ref[...] = out


def _stage4(h5, bt3, ps, fcW, fcb):
    return pl.pallas_call(
        _k4,
        grid=(NB,),
        in_specs=[
            pl.BlockSpec((BLK, D), lambda i: (i, 0)),
            pl.BlockSpec((1, 1, BLK), lambda i: (i, 0, 0)),
            pl.BlockSpec((L * G, D), lambda i: (0, 0)),
            pl.BlockSpec((L + 1, D, D), lambda i: (0, 0, 0)),
            pl.BlockSpec((L + 1, D), lambda i: (0, 0)),
        ],
        out_specs=pl.BlockSpec((G, D), lambda i: (0, 0)),
        out_shape=jax.ShapeDtypeStruct((G, D), jnp.float32),
        scratch_shapes=[
            pltpu.VMEM((G, D), jnp.float32),
            pltpu.VMEM((G, D), jnp.float32),
        ],
    )(h5, bt3, ps, fcW, fcb)


def kernel(x, edge_index, batch, convW1, convb1, bn1g, bn1b, convW2, convb2,
           bng, bnb, fcW, fcb):
    src = edge_index[0].reshape(NW, EW)
    dst = edge_index[1].reshape(NW, EW)
    pad = NCH * CH - EW
    src3 = jnp.concatenate(
        [src, jnp.zeros((NW, pad), jnp.int32)], axis=1).reshape(NW, NCH, CH)
    dst3 = jnp.concatenate(
        [dst, jnp.full((NW, pad), DUMMY, jnp.int32)], axis=1).reshape(
            NW, NCH, CH)
    zt = jnp.zeros((RPT, D), jnp.float32)
    bt3 = batch.reshape(NB, 1, BLK)

    h = x
    pooled = []
    for i in range(L):
        p = _sc_seg(h, src3, dst3, zt)
        pooled.append(_pool(h, bt3))
        z1, st1 = _stage1(h, p[0], p[1], convW1[i], convb1[i][None, :])
        z2, st2 = _stage2(z1, st1, convW2[i], convb2[i][None, :],
                          bn1g[i][None, :], bn1b[i][None, :])
        h = _stage3(z2, st2, bng[i][None, :], bnb[i][None, :])
    ps = jnp.concatenate(pooled, axis=0)
    return _stage4(h, bt3, ps, fcW, fcb)


# speedup vs baseline: 1.0218x; 1.0218x over previous
import functools

import jax
import jax.numpy as jnp
from jax import lax
from jax.experimental import pallas as pl
from jax.experimental.pallas import tpu as pltpu
from jax.experimental.pallas import tpu_sc as plsc

N = 10000
E = 320000
D = 128
L = 5
G = 64
BN_EPS = 1e-5

NW = 32
EW = E // NW
CH = 80
NCH = 128
WIN = 8
NWIN = NCH // WIN
NBUF = 4
LA = NBUF - 1
NPAD = 10112
RPT = NPAD // 16
DUMMY = N

BLK = 1000
NB = N // BLK

def _sc_body(h_hbm, src_hbm, dst_hbm, zt_hbm, out_hbm, *refs):
    sv = refs[0:2]
    dv = refs[2:4]
    bufs = refs[4:4 + NBUF]
    agg = refs[4 + NBUF]
    gsem = refs[5 + NBUF:5 + 2 * NBUF]
    asem = refs[5 + 2 * NBUF:5 + 3 * NBUF]
    isem = refs[5 + 3 * NBUF:7 + 3 * NBUF]
    jsem = refs[7 + 3 * NBUF:9 + 3 * NBUF]
    sv0, sv1 = sv
    dv0, dv1 = dv
    c = lax.axis_index("c")
    s = lax.axis_index("s")
    wk = c * 16 + s
    base = s * RPT
    pltpu.sync_copy(zt_hbm, bufs[0])
    for q in range(RPT // CH):
        pltpu.sync_copy(bufs[0], agg.at[pl.ds(base + q * CH, CH)])
    pltpu.sync_copy(bufs[0].at[pl.ds(0, RPT - (RPT // CH) * CH)],
                    agg.at[pl.ds(base + (RPT // CH) * CH,
                                 RPT - (RPT // CH) * CH)])
    pltpu.sync_copy(src_hbm.at[wk, pl.ds(0, WIN)], sv0)
    pltpu.sync_copy(src_hbm.at[wk, pl.ds(WIN, WIN)], sv1)
    pltpu.sync_copy(dst_hbm.at[wk, pl.ds(0, WIN)], dv0)
    pltpu.sync_copy(dst_hbm.at[wk, pl.ds(WIN, WIN)], dv1)
    plsc.subcore_barrier()

    for b in range(LA):
        pltpu.async_copy(h_hbm.at[sv0.at[b]], bufs[b], gsem[b])

    def window(w, p):
        for k in range(WIN):
            ch = w * WIN + k
            b = k % NBUF
            b3 = (k + LA) % NBUF
            pltpu.make_async_copy(
                h_hbm.at[sv[p].at[k]], bufs[b], gsem[b]).wait()
            if k == 0:
                @pl.when(w > 0)
                def _():
                    pltpu.make_async_copy(
                        bufs[b3], agg.at[dv[p].at[0]], asem[b3]).wait()

                @pl.when(jnp.logical_and(w > 0, w + 1 < NWIN))
                def _():
                    pltpu.async_copy(
                        src_hbm.at[wk, pl.ds((w + 1) * WIN, WIN)],
                        sv[1 - p], jsem[1 - p])
                    pltpu.async_copy(
                        dst_hbm.at[wk, pl.ds((w + 1) * WIN, WIN)],
                        dv[1 - p], isem[1 - p])

                @pl.when(w > 1)
                def _():
                    pltpu.make_async_copy(
                        dst_hbm.at[wk, pl.ds(w * WIN, WIN)], dv[p],
                        isem[p]).wait()
            else:
                pltpu.make_async_copy(
                    bufs[b3], agg.at[dv[p].at[k - 1]], asem[b3]).wait()

            if k == WIN - LA or (LA > WIN and k == 1):
                @pl.when(jnp.logical_and(w > 0, w + 1 < NWIN))
                def _():
                    pltpu.make_async_copy(
                        src_hbm.at[wk, pl.ds((w + 1) * WIN, WIN)],
                        sv[1 - p], jsem[1 - p]).wait()

            if k + LA < WIN:
                pltpu.async_copy(h_hbm.at[sv[p].at[k + LA]], bufs[b3],
                                 gsem[b3])
            else:
                @pl.when(w + 1 < NWIN)
                def _():
                    pltpu.async_copy(
                        h_hbm.at[sv[1 - p].at[k + LA - WIN]], bufs[b3],
                        gsem[b3])

            pltpu.async_copy(bufs[b], agg.at[dv[p].at[k]], asem[b], add=True)

    def wpair(wp, carry):
        window(wp * 2, 0)
        window(wp * 2 + 1, 1)
        return carry

    lax.fori_loop(0, NWIN // 2, wpair, 0)
    pltpu.make_async_copy(
        bufs[(NCH - 1) % NBUF], agg.at[dv1.at[WIN - 1]],
        asem[(NCH - 1) % NBUF]).wait()
    plsc.subcore_barrier()
    pltpu.sync_copy(agg.at[pl.ds(base, RPT)], out_hbm.at[c, pl.ds(base, RPT)])


@functools.cache
def _sc_seg_kernel():
    mesh = plsc.VectorSubcoreMesh(core_axis_name="c", subcore_axis_name="s")
    return pl.kernel(
        _sc_body,
        out_type=jax.ShapeDtypeStruct((2, NPAD, D), jnp.float32),
        mesh=mesh,
        scratch_types=(
            [pltpu.VMEM((WIN, CH), jnp.int32) for _ in range(4)]
            + [pltpu.VMEM((CH, D), jnp.float32) for _ in range(NBUF)]
            + [pltpu.VMEM_SHARED((NPAD, D), jnp.float32)]
            + [pltpu.SemaphoreType.DMA] * (2 * NBUF + 4)
        ),
    )


def _sc_seg(h, src3, dst3, zt):
    return _sc_seg_kernel()(h, src3, dst3, zt)


def _k1(h_ref, p0_ref, p1_ref, w_ref, b_ref, z_ref, st_ref):
    i = pl.program_id(0)
    sv = h_ref[...] + p0_ref[...] + p1_ref[...]
    z = jnp.dot(sv, w_ref[...], preferred_element_type=jnp.float32) + b_ref[...]
    z_ref[...] = z
    cs = jnp.sum(z, axis=0, keepdims=True)
    cq = jnp.sum(z * z, axis=0, keepdims=True)
    contrib = jnp.concatenate([cs, cq, jnp.zeros((6, D), jnp.float32)], axis=0)

    @pl.when(i == 0)
    def _():
        st_ref[...] = contrib

    @pl.when(i != 0)
    def _():
        st_ref[...] += contrib


def _stage1(h, p0, p1, W1, b1):
    return pl.pallas_call(
        _k1,
        grid=(NB,),
        in_specs=[
            pl.BlockSpec((BLK, D), lambda i: (i, 0)),
            pl.BlockSpec((BLK, D), lambda i: (i, 0)),
            pl.BlockSpec((BLK, D), lambda i: (i, 0)),
            pl.BlockSpec((D, D), lambda i: (0, 0)),
            pl.BlockSpec((1, D), lambda i: (0, 0)),
        ],
        out_specs=[
            pl.BlockSpec((BLK, D), lambda i: (i, 0)),
            pl.BlockSpec((8, D), lambda i: (0, 0)),
        ],
        out_shape=[
            jax.ShapeDtypeStruct((N, D), jnp.float32),
            jax.ShapeDtypeStruct((8, D), jnp.float32),
        ],
    )(h, p0, p1, W1, b1)


def _kpool(h_ref, bt_ref, pool_ref):
    i = pl.program_id(0)
    b = bt_ref[0, 0, :]
    oh = (lax.broadcasted_iota(jnp.int32, (G, BLK), 0) == b[None, :]).astype(
        jnp.float32)
    pc = jnp.dot(oh, h_ref[...], preferred_element_type=jnp.float32)

    @pl.when(i == 0)
    def _():
        pool_ref[...] = pc

    @pl.when(i != 0)
    def _():
        pool_ref[...] += pc


def _pool(h, bt3):
    return pl.pallas_call(
        _kpool,
        grid=(NB,),
        in_specs=[
            pl.BlockSpec((BLK, D), lambda i: (i, 0)),
            pl.BlockSpec((1, 1, BLK), lambda i: (i, 0, 0)),
        ],
        out_specs=pl.BlockSpec((G, D), lambda i: (0, 0)),
        out_shape=jax.ShapeDtypeStruct((G, D), jnp.float32),
    )(h, bt3)


def _k2(z_ref, st_ref, w_ref, b_ref, g_ref, bb_ref, o_ref, st2_ref):
    i = pl.program_id(0)
    st = st_ref[...]
    mu = st[0:1, :] * (1.0 / N)
    var = st[1:2, :] * (1.0 / N) - mu * mu
    scale = g_ref[...] * lax.rsqrt(var + BN_EPS)
    r = jnp.maximum((z_ref[...] - mu) * scale + bb_ref[...], 0.0)
    z2 = jnp.dot(r, w_ref[...], preferred_element_type=jnp.float32) + b_ref[...]
    o_ref[...] = z2
    cs = jnp.sum(z2, axis=0, keepdims=True)
    cq = jnp.sum(z2 * z2, axis=0, keepdims=True)
    contrib = jnp.concatenate([cs, cq, jnp.zeros((6, D), jnp.float32)], axis=0)

    @pl.when(i == 0)
    def _():
        st2_ref[...] = contrib

    @pl.when(i != 0)
    def _():
        st2_ref[...] += contrib


def _stage2(z1, st1, W2, b2, g1, bb1):
    return pl.pallas_call(
        _k2,
        grid=(NB,),
        in_specs=[
            pl.BlockSpec((BLK, D), lambda i: (i, 0)),
            pl.BlockSpec((8, D), lambda i: (0, 0)),
            pl.BlockSpec((D, D), lambda i: (0, 0)),
            pl.BlockSpec((1, D), lambda i: (0, 0)),
            pl.BlockSpec((1, D), lambda i: (0, 0)),
            pl.BlockSpec((1, D), lambda i: (0, 0)),
        ],
        out_specs=[
            pl.BlockSpec((BLK, D), lambda i: (i, 0)),
            pl.BlockSpec((8, D), lambda i: (0, 0)),
        ],
        out_shape=[
            jax.ShapeDtypeStruct((N, D), jnp.float32),
            jax.ShapeDtypeStruct((8, D), jnp.float32),
        ],
    )(z1, st1, W2, b2, g1, bb1)


def _k3(z_ref, st_ref, g_ref, bb_ref, h_ref):
    st = st_ref[...]
    mu = st[0:1, :] * (1.0 / N)
    var = st[1:2, :] * (1.0 / N) - mu * mu
    scale = g_ref[...] * lax.rsqrt(var + BN_EPS)
    h_ref[...] = jnp.maximum((z_ref[...] - mu) * scale + bb_ref[...], 0.0)


def _stage3(z2, st2, g, bb):
    return pl.pallas_call(
        _k3,
        grid=(NB,),
        in_specs=[
            pl.BlockSpec((BLK, D), lambda i: (i, 0)),
            pl.BlockSpec((8, D), lambda i: (0, 0)),
            pl.BlockSpec((1, D), lambda i: (0, 0)),
            pl.BlockSpec((1, D), lambda i: (0, 0)),
        ],
        out_specs=pl.BlockSpec((BLK, D), lambda i: (i, 0)),
        out_shape=jax.ShapeDtypeStruct((N, D), jnp.float32),
    )(z2, st2, g, bb)


def _k4(h_ref, bt_ref, ps_ref, fw_ref, fb_ref, o_ref, acc_ref, cnt_ref):
    i = pl.program_id(0)

    @pl.when(i == 0)
    def _():
        acc_ref[...] = jnp.zeros_like(acc_ref)
        cnt_ref[...] = jnp.zeros_like(cnt_ref)

    b = bt_ref[0, 0, :]
    oh = (lax.broadcasted_iota(jnp.int32, (G, BLK), 0) == b[None, :]).astype(
        jnp.float32)
    acc_ref[...] += jnp.dot(oh, h_ref[...], preferred_element_type=jnp.float32)
    cnt_ref[...] += jnp.dot(oh, jnp.ones((BLK, D), jnp.float32),
                            preferred_element_type=jnp.float32)

    @pl.when(i == NB - 1)
    def _():
        invc = 1.0 / jnp.maximum(cnt_ref[...], 1.0)
        out = jnp.dot(acc_ref[...] * invc, fw_ref[L],
                      preferred_element_type=jnp.float32)
        for k in range(L):
            out += jnp.dot(ps_ref[k * G:(k + 1) * G, :] * invc, fw_ref[k],
                           preferred_element_type=jnp.float32)
        out += jnp.sum(fb_ref[...], axis=0, keepdims=True)
        o_ref[...] = out


def _stage4(h5, bt3, ps, fcW, fcb):
    return pl.pallas_call(
        _k4,
        grid=(NB,),
        in_specs=[
            pl.BlockSpec((BLK, D), lambda i: (i, 0)),
            pl.BlockSpec((1, 1, BLK), lambda i: (i, 0, 0)),
            pl.BlockSpec((L * G, D), lambda i: (0, 0)),
            pl.BlockSpec((L + 1, D, D), lambda i: (0, 0, 0)),
            pl.BlockSpec((L + 1, D), lambda i: (0, 0)),
        ],
        out_specs=pl.BlockSpec((G, D), lambda i: (0, 0)),
        out_shape=jax.ShapeDtypeStruct((G, D), jnp.float32),
        scratch_shapes=[
            pltpu.VMEM((G, D), jnp.float32),
            pltpu.VMEM((G, D), jnp.float32),
        ],
    )(h5, bt3, ps, fcW, fcb)


def kernel(x, edge_index, batch, convW1, convb1, bn1g, bn1b, convW2, convb2,
           bng, bnb, fcW, fcb):
    src = edge_index[0].reshape(NW, EW)
    dst = edge_index[1].reshape(NW, EW)
    pad = NCH * CH - EW
    src3 = jnp.concatenate(
        [src, jnp.zeros((NW, pad), jnp.int32)], axis=1).reshape(NW, NCH, CH)
    dst3 = jnp.concatenate(
        [dst, jnp.full((NW, pad), DUMMY, jnp.int32)], axis=1).reshape(
            NW, NCH, CH)
    zt = jnp.zeros((CH, D), jnp.float32)
    bt3 = batch.reshape(NB, 1, BLK)

    h = x
    pooled = []
    for i in range(L):
        p = _sc_seg(h, src3, dst3, zt)
        pooled.append(_pool(h, bt3))
        z1, st1 = _stage1(h, p[0], p[1], convW1[i], convb1[i][None, :])
        z2, st2 = _stage2(z1, st1, convW2[i], convb2[i][None, :],
                          bn1g[i][None, :], bn1b[i][None, :])
        h = _stage3(z2, st2, bng[i][None, :], bnb[i][None, :])
    ps = jnp.concatenate(pooled, axis=0)
    return _stage4(h, bt3, ps, fcW, fcb)


# speedup vs baseline: 1.8589x; 1.8192x over previous
import functools

import jax
import jax.numpy as jnp
from jax import lax
from jax.experimental import pallas as pl
from jax.experimental.pallas import tpu as pltpu
from jax.experimental.pallas import tpu_sc as plsc

N = 10000
E = 320000
D = 128
L = 5
G = 64
BN_EPS = 1e-5

NW = 32
EW = E // NW
CH = 84
NCH = 120
WIN = 8
NWIN = NCH // WIN
NBUF = 4
LA = NBUF - 1
ZCH = 80
NPAD = 10112
RPT = NPAD // 16
DUMMY = N

BLK = 1000
NB = N // BLK

def _sc_body(h_hbm, src_hbm, dst_hbm, zt_hbm, out_hbm, *refs):
    sv = refs[0:2]
    dv = refs[2:4]
    bufs = refs[4:4 + NBUF]
    agg = refs[4 + NBUF]
    gsem = refs[5 + NBUF:5 + 2 * NBUF]
    asem = refs[5 + 2 * NBUF:5 + 3 * NBUF]
    isem = refs[5 + 3 * NBUF:7 + 3 * NBUF]
    jsem = refs[7 + 3 * NBUF:9 + 3 * NBUF]
    sv0, sv1 = sv
    dv0, dv1 = dv
    c = lax.axis_index("c")
    s = lax.axis_index("s")
    wk = c * 16 + s
    base = s * RPT
    pltpu.sync_copy(zt_hbm, bufs[0].at[pl.ds(0, ZCH)])
    for q in range(RPT // ZCH):
        pltpu.sync_copy(bufs[0].at[pl.ds(0, ZCH)],
                        agg.at[pl.ds(base + q * ZCH, ZCH)])
    pltpu.sync_copy(bufs[0].at[pl.ds(0, RPT - (RPT // ZCH) * ZCH)],
                    agg.at[pl.ds(base + (RPT // ZCH) * ZCH,
                                 RPT - (RPT // ZCH) * ZCH)])
    pltpu.sync_copy(src_hbm.at[wk, pl.ds(0, WIN)], sv0)
    pltpu.sync_copy(src_hbm.at[wk, pl.ds(WIN, WIN)], sv1)
    pltpu.sync_copy(dst_hbm.at[wk, pl.ds(0, WIN)], dv0)
    pltpu.sync_copy(dst_hbm.at[wk, pl.ds(WIN, WIN)], dv1)
    plsc.subcore_barrier()

    for b in range(LA):
        pltpu.async_copy(h_hbm.at[sv0.at[b]], bufs[b], gsem[b])

    def window(w, p):
        for k in range(WIN):
            ch = w * WIN + k
            b = k % NBUF
            b3 = (k + LA) % NBUF
            pltpu.make_async_copy(
                h_hbm.at[sv[p].at[k]], bufs[b], gsem[b]).wait()
            if k == 0:
                @pl.when(w > 0)
                def _():
                    pltpu.make_async_copy(
                        bufs[b3], agg.at[dv[p].at[0]], asem[b3]).wait()

                @pl.when(jnp.logical_and(w > 0, w + 1 < NWIN))
                def _():
                    pltpu.async_copy(
                        src_hbm.at[wk, pl.ds((w + 1) * WIN, WIN)],
                        sv[1 - p], jsem[1 - p])
                    pltpu.async_copy(
                        dst_hbm.at[wk, pl.ds((w + 1) * WIN, WIN)],
                        dv[1 - p], isem[1 - p])

                @pl.when(w > 1)
                def _():
                    pltpu.make_async_copy(
                        dst_hbm.at[wk, pl.ds(w * WIN, WIN)], dv[p],
                        isem[p]).wait()
            else:
                pltpu.make_async_copy(
                    bufs[b3], agg.at[dv[p].at[k - 1]], asem[b3]).wait()

            if k == WIN - LA or (LA > WIN and k == 1):
                @pl.when(jnp.logical_and(w > 0, w + 1 < NWIN))
                def _():
                    pltpu.make_async_copy(
                        src_hbm.at[wk, pl.ds((w + 1) * WIN, WIN)],
                        sv[1 - p], jsem[1 - p]).wait()

            if k + LA < WIN:
                pltpu.async_copy(h_hbm.at[sv[p].at[k + LA]], bufs[b3],
                                 gsem[b3])
            else:
                @pl.when(w + 1 < NWIN)
                def _():
                    pltpu.async_copy(
                        h_hbm.at[sv[1 - p].at[k + LA - WIN]], bufs[b3],
                        gsem[b3])

            pltpu.async_copy(bufs[b], agg.at[dv[p].at[k]], asem[b], add=True)

    def wpair(wp, carry):
        window(wp * 2, 0)
        window(wp * 2 + 1, 1)
        return carry

    lax.fori_loop(0, NWIN // 2, wpair, 0)
    if NWIN % 2:
        window(jnp.int32(NWIN - 1), (NWIN - 1) % 2)
    pltpu.make_async_copy(
        bufs[(NCH - 1) % NBUF], agg.at[dv1.at[WIN - 1]],
        asem[(NCH - 1) % NBUF]).wait()
    plsc.subcore_barrier()
    pltpu.sync_copy(agg.at[pl.ds(base, RPT)], out_hbm.at[c, pl.ds(base, RPT)])


@functools.cache
def _sc_seg_kernel():
    mesh = plsc.VectorSubcoreMesh(core_axis_name="c", subcore_axis_name="s")
    return pl.kernel(
        _sc_body,
        out_type=jax.ShapeDtypeStruct((2, NPAD, D), jnp.float32),
        mesh=mesh,
        scratch_types=(
            [pltpu.VMEM((WIN, CH), jnp.int32) for _ in range(4)]
            + [pltpu.VMEM((CH, D), jnp.float32) for _ in range(NBUF)]
            + [pltpu.VMEM_SHARED((NPAD, D), jnp.float32)]
            + [pltpu.SemaphoreType.DMA] * (2 * NBUF + 4)
        ),
    )


def _sc_seg(h, src3, dst3, zt):
    return _sc_seg_kernel()(h, src3, dst3, zt)


def _k1(h_ref, p0_ref, p1_ref, w_ref, b_ref, z_ref, st_ref):
    i = pl.program_id(0)
    sv = h_ref[...] + p0_ref[...] + p1_ref[...]
    z = jnp.dot(sv, w_ref[...], preferred_element_type=jnp.float32) + b_ref[...]
    z_ref[...] = z
    cs = jnp.sum(z, axis=0, keepdims=True)
    cq = jnp.sum(z * z, axis=0, keepdims=True)
    contrib = jnp.concatenate([cs, cq, jnp.zeros((6, D), jnp.float32)], axis=0)

    @pl.when(i == 0)
    def _():
        st_ref[...] = contrib

    @pl.when(i != 0)
    def _():
        st_ref[...] += contrib


def _stage1(h, p0, p1, W1, b1):
    return pl.pallas_call(
        _k1,
        grid=(NB,),
        in_specs=[
            pl.BlockSpec((BLK, D), lambda i: (i, 0)),
            pl.BlockSpec((BLK, D), lambda i: (i, 0)),
            pl.BlockSpec((BLK, D), lambda i: (i, 0)),
            pl.BlockSpec((D, D), lambda i: (0, 0)),
            pl.BlockSpec((1, D), lambda i: (0, 0)),
        ],
        out_specs=[
            pl.BlockSpec((BLK, D), lambda i: (i, 0)),
            pl.BlockSpec((8, D), lambda i: (0, 0)),
        ],
        out_shape=[
            jax.ShapeDtypeStruct((N, D), jnp.float32),
            jax.ShapeDtypeStruct((8, D), jnp.float32),
        ],
    )(h, p0, p1, W1, b1)


def _kpool(h_ref, bt_ref, pool_ref):
    i = pl.program_id(0)
    b = bt_ref[0, 0, :]
    oh = (lax.broadcasted_iota(jnp.int32, (G, BLK), 0) == b[None, :]).astype(
        jnp.float32)
    pc = jnp.dot(oh, h_ref[...], preferred_element_type=jnp.float32)

    @pl.when(i == 0)
    def _():
        pool_ref[...] = pc

    @pl.when(i != 0)
    def _():
        pool_ref[...] += pc


def _pool(h, bt3):
    return pl.pallas_call(
        _kpool,
        grid=(NB,),
        in_specs=[
            pl.BlockSpec((BLK, D), lambda i: (i, 0)),
            pl.BlockSpec((1, 1, BLK), lambda i: (i, 0, 0)),
        ],
        out_specs=pl.BlockSpec((G, D), lambda i: (0, 0)),
        out_shape=jax.ShapeDtypeStruct((G, D), jnp.float32),
    )(h, bt3)


def _k2(z_ref, st_ref, w_ref, b_ref, g_ref, bb_ref, o_ref, st2_ref):
    i = pl.program_id(0)
    st = st_ref[...]
    mu = st[0:1, :] * (1.0 / N)
    var = st[1:2, :] * (1.0 / N) - mu * mu
    scale = g_ref[...] * lax.rsqrt(var + BN_EPS)
    r = jnp.maximum((z_ref[...] - mu) * scale + bb_ref[...], 0.0)
    z2 = jnp.dot(r, w_ref[...], preferred_element_type=jnp.float32) + b_ref[...]
    o_ref[...] = z2
    cs = jnp.sum(z2, axis=0, keepdims=True)
    cq = jnp.sum(z2 * z2, axis=0, keepdims=True)
    contrib = jnp.concatenate([cs, cq, jnp.zeros((6, D), jnp.float32)], axis=0)

    @pl.when(i == 0)
    def _():
        st2_ref[...] = contrib

    @pl.when(i != 0)
    def _():
        st2_ref[...] += contrib


def _stage2(z1, st1, W2, b2, g1, bb1):
    return pl.pallas_call(
        _k2,
        grid=(NB,),
        in_specs=[
            pl.BlockSpec((BLK, D), lambda i: (i, 0)),
            pl.BlockSpec((8, D), lambda i: (0, 0)),
            pl.BlockSpec((D, D), lambda i: (0, 0)),
            pl.BlockSpec((1, D), lambda i: (0, 0)),
            pl.BlockSpec((1, D), lambda i: (0, 0)),
            pl.BlockSpec((1, D), lambda i: (0, 0)),
        ],
        out_specs=[
            pl.BlockSpec((BLK, D), lambda i: (i, 0)),
            pl.BlockSpec((8, D), lambda i: (0, 0)),
        ],
        out_shape=[
            jax.ShapeDtypeStruct((N, D), jnp.float32),
            jax.ShapeDtypeStruct((8, D), jnp.float32),
        ],
    )(z1, st1, W2, b2, g1, bb1)


def _k3(z_ref, st_ref, g_ref, bb_ref, h_ref):
    st = st_ref[...]
    mu = st[0:1, :] * (1.0 / N)
    var = st[1:2, :] * (1.0 / N) - mu * mu
    scale = g_ref[...] * lax.rsqrt(var + BN_EPS)
    h_ref[...] = jnp.maximum((z_ref[...] - mu) * scale + bb_ref[...], 0.0)


def _stage3(z2, st2, g, bb):
    return pl.pallas_call(
        _k3,
        grid=(NB,),
        in_specs=[
            pl.BlockSpec((BLK, D), lambda i: (i, 0)),
            pl.BlockSpec((8, D), lambda i: (0, 0)),
            pl.BlockSpec((1, D), lambda i: (0, 0)),
            pl.BlockSpec((1, D), lambda i: (0, 0)),
        ],
        out_specs=pl.BlockSpec((BLK, D), lambda i: (i, 0)),
        out_shape=jax.ShapeDtypeStruct((N, D), jnp.float32),
    )(z2, st2, g, bb)


def _k4(h_ref, bt_ref, ps_ref, fw_ref, fb_ref, o_ref, acc_ref, cnt_ref):
    i = pl.program_id(0)

    @pl.when(i == 0)
    def _():
        acc_ref[...] = jnp.zeros_like(acc_ref)
        cnt_ref[...] = jnp.zeros_like(cnt_ref)

    b = bt_ref[0, 0, :]
    oh = (lax.broadcasted_iota(jnp.int32, (G, BLK), 0) == b[None, :]).astype(
        jnp.float32)
    acc_ref[...] += jnp.dot(oh, h_ref[...], preferred_element_type=jnp.float32)
    cnt_ref[...] += jnp.dot(oh, jnp.ones((BLK, D), jnp.float32),
                            preferred_element_type=jnp.float32)

    @pl.when(i == NB - 1)
    def _():
        invc = 1.0 / jnp.maximum(cnt_ref[...], 1.0)
        out = jnp.dot(acc_ref[...] * invc, fw_ref[L],
                      preferred_element_type=jnp.float32)
        for k in range(L):
            out += jnp.dot(ps_ref[k * G:(k + 1) * G, :] * invc, fw_ref[k],
                           preferred_element_type=jnp.float32)
        out += jnp.sum(fb_ref[...], axis=0, keepdims=True)
        o_ref[...] = out


def _stage4(h5, bt3, ps, fcW, fcb):
    return pl.pallas_call(
        _k4,
        grid=(NB,),
        in_specs=[
            pl.BlockSpec((BLK, D), lambda i: (i, 0)),
            pl.BlockSpec((1, 1, BLK), lambda i: (i, 0, 0)),
            pl.BlockSpec((L * G, D), lambda i: (0, 0)),
            pl.BlockSpec((L + 1, D, D), lambda i: (0, 0, 0)),
            pl.BlockSpec((L + 1, D), lambda i: (0, 0)),
        ],
        out_specs=pl.BlockSpec((G, D), lambda i: (0, 0)),
        out_shape=jax.ShapeDtypeStruct((G, D), jnp.float32),
        scratch_shapes=[
            pltpu.VMEM((G, D), jnp.float32),
            pltpu.VMEM((G, D), jnp.float32),
        ],
    )(h5, bt3, ps, fcW, fcb)


def kernel(x, edge_index, batch, convW1, convb1, bn1g, bn1b, convW2, convb2,
           bng, bnb, fcW, fcb):
    src = edge_index[0].reshape(NW, EW)
    dst = edge_index[1].reshape(NW, EW)
    pad = NCH * CH - EW
    src3 = jnp.concatenate(
        [src, jnp.zeros((NW, pad), jnp.int32)], axis=1).reshape(NW, NCH, CH)
    dst3 = jnp.concatenate(
        [dst, jnp.full((NW, pad), DUMMY, jnp.int32)], axis=1).reshape(
            NW, NCH, CH)
    zt = jnp.zeros((ZCH, D), jnp.float32)
    bt3 = batch.reshape(NB, 1, BLK)

    h = x
    pooled = []
    for i in range(L):
        p = _sc_seg(h, src3, dst3, zt)
        pooled.append(_pool(h, bt3))
        z1, st1 = _stage1(h, p[0], p[1], convW1[i], convb1[i][None, :])
        z2, st2 = _stage2(z1, st1, convW2[i], convb2[i][None, :],
                          bn1g[i][None, :], bn1b[i][None, :])
        h = _stage3(z2, st2, bng[i][None, :], bnb[i][None, :])
    ps = jnp.concatenate(pooled, axis=0)
    return _stage4(h, bt3, ps, fcW, fcb)


# speedup vs baseline: 1.9396x; 1.0434x over previous
import functools

import jax
import jax.numpy as jnp
from jax import lax
from jax.experimental import pallas as pl
from jax.experimental.pallas import tpu as pltpu
from jax.experimental.pallas import tpu_sc as plsc

N = 10000
E = 320000
D = 128
L = 5
G = 64
BN_EPS = 1e-5

NW = 32
EW = E // NW
CH = 84
NCH = 120
WIN = 8
NWIN = NCH // WIN
NBUF = 4
LA = NBUF - 1
ZCH = 80
NPAD = 10112
RPT = NPAD // 16
DUMMY = N

BLK = 1000
NB = N // BLK

def _sc_body(h_hbm, src_hbm, dst_hbm, zt_hbm, out_hbm, *refs):
    sv = refs[0:2]
    dv = refs[2:4]
    bufs = refs[4:4 + NBUF]
    agg = refs[4 + NBUF]
    gsem = refs[5 + NBUF:5 + 2 * NBUF]
    asem = refs[5 + 2 * NBUF:5 + 3 * NBUF]
    isem = refs[5 + 3 * NBUF:7 + 3 * NBUF]
    jsem = refs[7 + 3 * NBUF:9 + 3 * NBUF]
    sv0, sv1 = sv
    dv0, dv1 = dv
    c = lax.axis_index("c")
    s = lax.axis_index("s")
    wk = c * 16 + s
    base = s * RPT
    pltpu.sync_copy(zt_hbm, bufs[0].at[pl.ds(0, ZCH)])
    for q in range(RPT // ZCH):
        pltpu.sync_copy(bufs[0].at[pl.ds(0, ZCH)],
                        agg.at[pl.ds(base + q * ZCH, ZCH)])
    pltpu.sync_copy(bufs[0].at[pl.ds(0, RPT - (RPT // ZCH) * ZCH)],
                    agg.at[pl.ds(base + (RPT // ZCH) * ZCH,
                                 RPT - (RPT // ZCH) * ZCH)])
    pltpu.sync_copy(src_hbm.at[wk, pl.ds(0, WIN)], sv0)
    pltpu.sync_copy(src_hbm.at[wk, pl.ds(WIN, WIN)], sv1)
    pltpu.sync_copy(dst_hbm.at[wk, pl.ds(0, WIN)], dv0)
    pltpu.sync_copy(dst_hbm.at[wk, pl.ds(WIN, WIN)], dv1)
    plsc.subcore_barrier()

    for b in range(LA):
        pltpu.async_copy(h_hbm.at[sv0.at[b]], bufs[b], gsem[b])

    def window(w, p):
        for k in range(WIN):
            ch = w * WIN + k
            b = k % NBUF
            b3 = (k + LA) % NBUF
            pltpu.make_async_copy(
                h_hbm.at[sv[p].at[k]], bufs[b], gsem[b]).wait()
            if k == 0:
                @pl.when(w > 0)
                def _():
                    pltpu.make_async_copy(
                        bufs[b3], agg.at[dv[p].at[0]], asem[b3]).wait()

                @pl.when(jnp.logical_and(w > 0, w + 1 < NWIN))
                def _():
                    pltpu.async_copy(
                        src_hbm.at[wk, pl.ds((w + 1) * WIN, WIN)],
                        sv[1 - p], jsem[1 - p])
                    pltpu.async_copy(
                        dst_hbm.at[wk, pl.ds((w + 1) * WIN, WIN)],
                        dv[1 - p], isem[1 - p])

                @pl.when(w > 1)
                def _():
                    pltpu.make_async_copy(
                        dst_hbm.at[wk, pl.ds(w * WIN, WIN)], dv[p],
                        isem[p]).wait()
            else:
                pltpu.make_async_copy(
                    bufs[b3], agg.at[dv[p].at[k - 1]], asem[b3]).wait()

            if k == WIN - LA or (LA > WIN and k == 1):
                @pl.when(jnp.logical_and(w > 0, w + 1 < NWIN))
                def _():
                    pltpu.make_async_copy(
                        src_hbm.at[wk, pl.ds((w + 1) * WIN, WIN)],
                        sv[1 - p], jsem[1 - p]).wait()

            if k + LA < WIN:
                pltpu.async_copy(h_hbm.at[sv[p].at[k + LA]], bufs[b3],
                                 gsem[b3])
            else:
                @pl.when(w + 1 < NWIN)
                def _():
                    pltpu.async_copy(
                        h_hbm.at[sv[1 - p].at[k + LA - WIN]], bufs[b3],
                        gsem[b3])

            pltpu.async_copy(bufs[b], agg.at[dv[p].at[k]], asem[b], add=True)

    def wpair(wp, carry):
        window(wp * 2, 0)
        window(wp * 2 + 1, 1)
        return carry

    lax.fori_loop(0, NWIN // 2, wpair, 0)
    if NWIN % 2:
        window(jnp.int32(NWIN - 1), (NWIN - 1) % 2)
    pltpu.make_async_copy(
        bufs[(NCH - 1) % NBUF], agg.at[dv1.at[WIN - 1]],
        asem[(NCH - 1) % NBUF]).wait()
    plsc.subcore_barrier()
    pltpu.sync_copy(agg.at[pl.ds(base, RPT)], out_hbm.at[c, pl.ds(base, RPT)])


@functools.cache
def _sc_seg_kernel():
    mesh = plsc.VectorSubcoreMesh(core_axis_name="c", subcore_axis_name="s")
    return pl.kernel(
        _sc_body,
        out_type=jax.ShapeDtypeStruct((2, NPAD, D), jnp.float32),
        mesh=mesh,
        scratch_types=(
            [pltpu.VMEM((WIN, CH), jnp.int32) for _ in range(4)]
            + [pltpu.VMEM((CH, D), jnp.float32) for _ in range(NBUF)]
            + [pltpu.VMEM_SHARED((NPAD, D), jnp.float32)]
            + [pltpu.SemaphoreType.DMA] * (2 * NBUF + 4)
        ),
    )


def _sc_seg(h, src3, dst3, zt):
    return _sc_seg_kernel()(h, src3, dst3, zt)


def _k123(h_ref, p0_ref, p1_ref, w1_ref, b1_ref, w2_ref, b2_ref,
          g1_ref, bb1_ref, g_ref, bb_ref, h_out,
          z1_buf, z2_buf, st1, st2):
    ph = pl.program_id(0)
    i = pl.program_id(1)

    @pl.when(jnp.logical_and(ph == 0, i == 0))
    def _():
        st1[...] = jnp.zeros_like(st1)
        st2[...] = jnp.zeros_like(st2)

    @pl.when(ph == 0)
    def _():
        sv = h_ref[...] + p0_ref[...] + p1_ref[...]
        z = jnp.dot(sv, w1_ref[...],
                    preferred_element_type=jnp.float32) + b1_ref[...]
        z1_buf[pl.ds(i * BLK, BLK), :] = z
        cs = jnp.sum(z, axis=0, keepdims=True)
        cq = jnp.sum(z * z, axis=0, keepdims=True)
        st1[...] += jnp.concatenate(
            [cs, cq, jnp.zeros((6, D), jnp.float32)], axis=0)

    @pl.when(ph == 1)
    def _():
        st = st1[...]
        mu = st[0:1, :] * (1.0 / N)
        var = st[1:2, :] * (1.0 / N) - mu * mu
        scale = g1_ref[...] * lax.rsqrt(var + BN_EPS)
        z1 = z1_buf[pl.ds(i * BLK, BLK), :]
        r = jnp.maximum((z1 - mu) * scale + bb1_ref[...], 0.0)
        z2 = jnp.dot(r, w2_ref[...],
                     preferred_element_type=jnp.float32) + b2_ref[...]
        z2_buf[pl.ds(i * BLK, BLK), :] = z2
        cs = jnp.sum(z2, axis=0, keepdims=True)
        cq = jnp.sum(z2 * z2, axis=0, keepdims=True)
        st2[...] += jnp.concatenate(
            [cs, cq, jnp.zeros((6, D), jnp.float32)], axis=0)

    @pl.when(ph == 2)
    def _():
        st = st2[...]
        mu = st[0:1, :] * (1.0 / N)
        var = st[1:2, :] * (1.0 / N) - mu * mu
        scale = g_ref[...] * lax.rsqrt(var + BN_EPS)
        z2 = z2_buf[pl.ds(i * BLK, BLK), :]
        h_out[...] = jnp.maximum((z2 - mu) * scale + bb_ref[...], 0.0)


def _stage123(h, p0, p1, W1, b1, W2, b2, g1, bb1, g, bb):
    blk_ph0 = lambda ph, i: (jnp.where(ph == 0, i, 0), 0)
    const2 = lambda ph, i: (0, 0)
    return pl.pallas_call(
        _k123,
        grid=(3, NB),
        in_specs=[
            pl.BlockSpec((BLK, D), blk_ph0),
            pl.BlockSpec((BLK, D), blk_ph0),
            pl.BlockSpec((BLK, D), blk_ph0),
            pl.BlockSpec((D, D), const2),
            pl.BlockSpec((1, D), const2),
            pl.BlockSpec((D, D), const2),
            pl.BlockSpec((1, D), const2),
            pl.BlockSpec((1, D), const2),
            pl.BlockSpec((1, D), const2),
            pl.BlockSpec((1, D), const2),
            pl.BlockSpec((1, D), const2),
        ],
        out_specs=pl.BlockSpec(
            (BLK, D), lambda ph, i: (jnp.where(ph == 2, i, 0), 0)),
        out_shape=jax.ShapeDtypeStruct((N, D), jnp.float32),
        scratch_shapes=[
            pltpu.VMEM((N, D), jnp.float32),
            pltpu.VMEM((N, D), jnp.float32),
            pltpu.VMEM((8, D), jnp.float32),
            pltpu.VMEM((8, D), jnp.float32),
        ],
    )(h, p0, p1, W1, b1, W2, b2, g1, bb1, g, bb)


def _kpool(h_ref, bt_ref, pool_ref):
    i = pl.program_id(0)
    b = bt_ref[0, 0, :]
    oh = (lax.broadcasted_iota(jnp.int32, (G, BLK), 0) == b[None, :]).astype(
        jnp.float32)
    pc = jnp.dot(oh, h_ref[...], preferred_element_type=jnp.float32)

    @pl.when(i == 0)
    def _():
        pool_ref[...] = pc

    @pl.when(i != 0)
    def _():
        pool_ref[...] += pc


def _pool(h, bt3):
    return pl.pallas_call(
        _kpool,
        grid=(NB,),
        in_specs=[
            pl.BlockSpec((BLK, D), lambda i: (i, 0)),
            pl.BlockSpec((1, 1, BLK), lambda i: (i, 0, 0)),
        ],
        out_specs=pl.BlockSpec((G, D), lambda i: (0, 0)),
        out_shape=jax.ShapeDtypeStruct((G, D), jnp.float32),
    )(h, bt3)


def _k4(h_ref, bt_ref, ps_ref, fw_ref, fb_ref, o_ref, acc_ref, cnt_ref):
    i = pl.program_id(0)

    @pl.when(i == 0)
    def _():
        acc_ref[...] = jnp.zeros_like(acc_ref)
        cnt_ref[...] = jnp.zeros_like(cnt_ref)

    b = bt_ref[0, 0, :]
    oh = (lax.broadcasted_iota(jnp.int32, (G, BLK), 0) == b[None, :]).astype(
        jnp.float32)
    acc_ref[...] += jnp.dot(oh, h_ref[...], preferred_element_type=jnp.float32)
    cnt_ref[...] += jnp.dot(oh, jnp.ones((BLK, D), jnp.float32),
                            preferred_element_type=jnp.float32)

    @pl.when(i == NB - 1)
    def _():
        invc = 1.0 / jnp.maximum(cnt_ref[...], 1.0)
        out = jnp.dot(acc_ref[...] * invc, fw_ref[L],
                      preferred_element_type=jnp.float32)
        for k in range(L):
            out += jnp.dot(ps_ref[k * G:(k + 1) * G, :] * invc, fw_ref[k],
                           preferred_element_type=jnp.float32)
        out += jnp.sum(fb_ref[...], axis=0, keepdims=True)
        o_ref[...] = out


def _stage4(h5, bt3, ps, fcW, fcb):
    return pl.pallas_call(
        _k4,
        grid=(NB,),
        in_specs=[
            pl.BlockSpec((BLK, D), lambda i: (i, 0)),
            pl.BlockSpec((1, 1, BLK), lambda i: (i, 0, 0)),
            pl.BlockSpec((L * G, D), lambda i: (0, 0)),
            pl.BlockSpec((L + 1, D, D), lambda i: (0, 0, 0)),
            pl.BlockSpec((L + 1, D), lambda i: (0, 0)),
        ],
        out_specs=pl.BlockSpec((G, D), lambda i: (0, 0)),
        out_shape=jax.ShapeDtypeStruct((G, D), jnp.float32),
        scratch_shapes=[
            pltpu.VMEM((G, D), jnp.float32),
            pltpu.VMEM((G, D), jnp.float32),
        ],
    )(h5, bt3, ps, fcW, fcb)


def kernel(x, edge_index, batch, convW1, convb1, bn1g, bn1b, convW2, convb2,
           bng, bnb, fcW, fcb):
    src = edge_index[0].reshape(NW, EW)
    dst = edge_index[1].reshape(NW, EW)
    pad = NCH * CH - EW
    src3 = jnp.concatenate(
        [src, jnp.zeros((NW, pad), jnp.int32)], axis=1).reshape(NW, NCH, CH)
    dst3 = jnp.concatenate(
        [dst, jnp.full((NW, pad), DUMMY, jnp.int32)], axis=1).reshape(
            NW, NCH, CH)
    zt = jnp.zeros((ZCH, D), jnp.float32)
    bt3 = batch.reshape(NB, 1, BLK)

    h = x
    pooled = []
    for i in range(L):
        p = _sc_seg(h, src3, dst3, zt)
        pooled.append(_pool(h, bt3))
        h = _stage123(h, p[0], p[1], convW1[i], convb1[i][None, :],
                      convW2[i], convb2[i][None, :],
                      bn1g[i][None, :], bn1b[i][None, :],
                      bng[i][None, :], bnb[i][None, :])
    ps = jnp.concatenate(pooled, axis=0)
    return _stage4(h, bt3, ps, fcW, fcb)
